# Initial kernel scaffold; baseline (speedup 1.0000x reference)
import functools

import jax
import jax.numpy as jnp
from jax.experimental import pallas as pl
from jax.experimental.pallas import tpu as pltpu
from jax.experimental.pallas import tpu_sc as plsc

_GATHER_WINDOW = 128
_BB = 64


def _sc_gather(table, idx_flat):
    n = idx_flat.shape[0]
    d = table.shape[1]
    idx2d = idx_flat.reshape(1, n)
    mesh = plsc.VectorSubcoreMesh(core_axis_name="core", subcore_axis_name="subcore")

    @functools.partial(
        pl.kernel,
        out_type=jax.ShapeDtypeStruct((n, d), table.dtype),
        mesh=mesh,
    )
    def k(x_hbm, i_hbm, o_hbm):
        def body(i_vmem, o_vmem):
            pltpu.sync_copy(x_hbm.at[i_vmem.at[0]], o_vmem)

        pltpu.emit_pipeline(
            body,
            grid=(n // _GATHER_WINDOW,),
            in_specs=[pl.BlockSpec((1, _GATHER_WINDOW), index_map=lambda i: (0, i))],
            out_specs=[pl.BlockSpec((_GATHER_WINDOW, d), index_map=lambda i: (i, 0))],
            core_axis_name=("core", "subcore"),
            dimension_semantics=(pltpu.PARALLEL,),
        )(i_hbm, o_hbm)

    return k(table, idx2d)


def _tc_body(gent, grec, rels, rel_emb, w1, w2, w3, wagg, bmat, out):
    d = gent.shape[-1]
    s = gent.shape[2]
    bb = gent.shape[1]
    w1a = w1[0:d, :]
    w1b = w1[d : 2 * d, :]
    r1tab = jnp.dot(rel_emb[...], w1b, preferred_element_type=jnp.float32)
    w3r = jnp.broadcast_to(w3[...], (d, d))
    b1 = bmat[0:1, :]
    b2 = bmat[1:2, :]
    bagg = bmat[2:3, :]
    b3 = bmat[3:4, 0:1]
    nr = rel_emb.shape[0]
    lane_iota = jax.lax.broadcasted_iota(jnp.int32, (1, nr), 1)

    def layer(h3, t3, rl):
        hf = h3.reshape(bb * s, d)
        oh = (rl == lane_iota).astype(jnp.float32)
        x = (jnp.dot(hf, w1a, preferred_element_type=jnp.float32)
             + jnp.dot(oh, r1tab, preferred_element_type=jnp.float32) + b1)
        x = jnp.maximum(x, 0.0)
        x = jnp.maximum(jnp.dot(x, w2, preferred_element_type=jnp.float32) + b2, 0.0)
        lg = jax.nn.sigmoid(jnp.dot(x, w3r, preferred_element_type=jnp.float32) + b3)
        p = jnp.exp(lg.reshape(bb, s, d))
        p = p / jnp.sum(p, axis=1, keepdims=True)
        return jnp.sum(p * t3, axis=1)

    def side(tab0, h0, h1, t0, t1, r0, r1):
        e0 = jnp.mean(tab0, axis=1)
        e1 = layer(h0, t0, r0)
        e2 = layer(h1, t1, r1)
        y = (jnp.dot(e0, wagg[0:d, :], preferred_element_type=jnp.float32)
             + jnp.dot(e1, wagg[d : 2 * d, :], preferred_element_type=jnp.float32)
             + jnp.dot(e2, wagg[2 * d : 3 * d, :], preferred_element_type=jnp.float32)
             + bagg)
        return jax.nn.sigmoid(y)

    ge = gent[...]
    rr = rels[...]
    u = side(grec[...], ge[0], ge[1], ge[2], ge[3], rr[0], rr[1])
    v = side(ge[8], ge[4], ge[5], ge[6], ge[7], rr[2], rr[3])
    out[...] = jax.nn.sigmoid(jnp.sum(u * v, axis=1, keepdims=True))


def _tc_specs(b, s, d, nr):
    bb = _BB
    grid = (b // bb,)
    in_specs = [
        pl.BlockSpec((9, bb, s, d), lambda i: (0, i, 0, 0)),
        pl.BlockSpec((bb, s, d), lambda i: (i, 0, 0)),
        pl.BlockSpec((4, bb * s, 1), lambda i: (0, i, 0)),
        pl.BlockSpec((nr, d), lambda i: (0, 0)),
        pl.BlockSpec((2 * d, d), lambda i: (0, 0)),
        pl.BlockSpec((d, d), lambda i: (0, 0)),
        pl.BlockSpec((d, 1), lambda i: (0, 0)),
        pl.BlockSpec((3 * d, d), lambda i: (0, 0)),
        pl.BlockSpec((8, d), lambda i: (0, 0)),
    ]
    out_specs = pl.BlockSpec((bb, 1), lambda i: (i, 0))
    out_shape = jax.ShapeDtypeStruct((b, 1), jnp.float32)
    return grid, in_specs, out_specs, out_shape


def kernel(u_entities, u_heads, u_relations, u_tails,
           v_entities, v_heads, v_relations, v_tails,
           entity_emb, rec_emb, rel_emb,
           W1, b1, W2, b2, W3, b3, Wagg, bagg):
    b, s = u_entities.shape
    d = entity_emb.shape[1]
    nr = rel_emb.shape[0]

    idx_ent = jnp.concatenate([
        u_heads.reshape(-1), u_tails.reshape(-1),
        v_heads.reshape(-1), v_tails.reshape(-1),
        v_entities.reshape(-1),
    ])
    g_ent = _sc_gather(entity_emb, idx_ent).reshape(9, b, s, d)
    g_rec = _sc_gather(rec_emb, u_entities.reshape(-1)).reshape(b, s, d)

    rels = jnp.concatenate([u_relations, v_relations], axis=0).reshape(4, b * s, 1)
    bmat = (jnp.zeros((8, d), jnp.float32)
            .at[0].set(b1).at[1].set(b2).at[2].set(bagg).at[3].set(b3[0]))

    grid, in_specs, out_specs, out_shape = _tc_specs(b, s, d, nr)
    out = pl.pallas_call(
        _tc_body,
        grid=grid,
        in_specs=in_specs,
        out_specs=out_specs,
        out_shape=out_shape,
    )(g_ent, g_rec, rels, rel_emb, W1, W2, W3, Wagg, bmat)
    return out.reshape(-1)


# speedup vs baseline: 3.5514x; 3.5514x over previous
import functools

import jax
import jax.numpy as jnp
from jax.experimental import pallas as pl
from jax.experimental.pallas import tpu as pltpu
from jax.experimental.pallas import tpu_sc as plsc

_GATHER_WINDOW = 128
_BB = 32


def _sc_gather(table, idx_flat):
    n = idx_flat.shape[0]
    d = table.shape[1]
    idx2d = idx_flat.reshape(1, n)
    mesh = plsc.VectorSubcoreMesh(core_axis_name="core", subcore_axis_name="subcore")

    @functools.partial(
        pl.kernel,
        out_type=jax.ShapeDtypeStruct((n, d), table.dtype),
        mesh=mesh,
        compiler_params=pltpu.CompilerParams(use_tc_tiling_on_sc=False),
    )
    def k(x_hbm, i_hbm, o_hbm):
        def body(i_vmem, o_vmem):
            pltpu.sync_copy(x_hbm.at[i_vmem.at[0]], o_vmem)

        pltpu.emit_pipeline(
            body,
            grid=(n // _GATHER_WINDOW,),
            in_specs=[pl.BlockSpec((1, _GATHER_WINDOW), index_map=lambda i: (0, i))],
            out_specs=[pl.BlockSpec((_GATHER_WINDOW, d), index_map=lambda i: (i, 0))],
            core_axis_name=("core", "subcore"),
            dimension_semantics=(pltpu.PARALLEL,),
        )(i_hbm, o_hbm)

    return k(table, idx2d)


def _tc_body(gent, grec, rels, rel_emb, w1, w2, w3, wagg, bmat, out):
    d = gent.shape[-1]
    s = gent.shape[2]
    bb = gent.shape[1]
    w1f = w1[...]
    waggf = wagg[...]
    bmatf = bmat[...]
    w1a = w1f[0:d, :]
    w1b = w1f[d : 2 * d, :]
    r1tab = jnp.dot(rel_emb[...], w1b, preferred_element_type=jnp.float32)
    w3r = jnp.broadcast_to(w3[...], (d, d))
    w2f = w2[...]
    b1 = bmatf[0:1, :]
    b2 = bmatf[1:2, :]
    bagg = bmatf[2:3, :]
    b3 = bmatf[3:4, 0:1]
    nr = rel_emb.shape[0]
    lane_iota = jax.lax.broadcasted_iota(jnp.int32, (1, nr), 1)

    def layer(h3, t3, rl):
        hf = h3.reshape(bb * s, d)
        oh = (rl == lane_iota).astype(jnp.float32)
        x = (jnp.dot(hf, w1a, preferred_element_type=jnp.float32)
             + jnp.dot(oh, r1tab, preferred_element_type=jnp.float32) + b1)
        x = jnp.maximum(x, 0.0)
        x = jnp.maximum(jnp.dot(x, w2f, preferred_element_type=jnp.float32) + b2, 0.0)
        lg = jax.nn.sigmoid(jnp.dot(x, w3r, preferred_element_type=jnp.float32) + b3)
        p = jnp.exp(lg.reshape(bb, s, d))
        p = p / jnp.sum(p, axis=1, keepdims=True)
        return jnp.sum(p * t3, axis=1)

    def side(tab0, h0, h1, t0, t1, r0, r1):
        e0 = jnp.mean(tab0, axis=1)
        e1 = layer(h0, t0, r0)
        e2 = layer(h1, t1, r1)
        y = (jnp.dot(e0, waggf[0:d, :], preferred_element_type=jnp.float32)
             + jnp.dot(e1, waggf[d : 2 * d, :], preferred_element_type=jnp.float32)
             + jnp.dot(e2, waggf[2 * d : 3 * d, :], preferred_element_type=jnp.float32)
             + bagg)
        return jax.nn.sigmoid(y)

    ge = [gent[i] for i in range(9)]
    rr = [rels[i] for i in range(4)]
    u = side(grec[...], ge[0], ge[1], ge[2], ge[3], rr[0], rr[1])
    v = side(ge[8], ge[4], ge[5], ge[6], ge[7], rr[2], rr[3])
    out[...] = jax.nn.sigmoid(jnp.sum(u * v, axis=1, keepdims=True))


def _tc_specs(b, s, d, nr):
    bb = _BB
    grid = (b // bb,)
    in_specs = [
        pl.BlockSpec((9, bb, s, d), lambda i: (0, i, 0, 0)),
        pl.BlockSpec((bb, s, d), lambda i: (i, 0, 0)),
        pl.BlockSpec((4, bb * s, 1), lambda i: (0, i, 0)),
        pl.BlockSpec((nr, d), lambda i: (0, 0)),
        pl.BlockSpec((2 * d, d), lambda i: (0, 0)),
        pl.BlockSpec((d, d), lambda i: (0, 0)),
        pl.BlockSpec((d, 1), lambda i: (0, 0)),
        pl.BlockSpec((3 * d, d), lambda i: (0, 0)),
        pl.BlockSpec((8, d), lambda i: (0, 0)),
    ]
    out_specs = pl.BlockSpec((bb, 1), lambda i: (i, 0))
    out_shape = jax.ShapeDtypeStruct((b, 1), jnp.float32)
    return grid, in_specs, out_specs, out_shape


def kernel(u_entities, u_heads, u_relations, u_tails,
           v_entities, v_heads, v_relations, v_tails,
           entity_emb, rec_emb, rel_emb,
           W1, b1, W2, b2, W3, b3, Wagg, bagg):
    b, s = u_entities.shape
    d = entity_emb.shape[1]
    nr = rel_emb.shape[0]

    idx_ent = jnp.concatenate([
        u_heads.reshape(-1), u_tails.reshape(-1),
        v_heads.reshape(-1), v_tails.reshape(-1),
        v_entities.reshape(-1),
    ])
    g_ent = _sc_gather(entity_emb, idx_ent).reshape(9, b, s, d)
    g_rec = _sc_gather(rec_emb, u_entities.reshape(-1)).reshape(b, s, d)

    rels = jnp.concatenate([u_relations, v_relations], axis=0).reshape(4, b * s, 1)
    bmat = (jnp.zeros((8, d), jnp.float32)
            .at[0].set(b1).at[1].set(b2).at[2].set(bagg).at[3].set(b3[0]))

    grid, in_specs, out_specs, out_shape = _tc_specs(b, s, d, nr)
    out = pl.pallas_call(
        _tc_body,
        grid=grid,
        in_specs=in_specs,
        out_specs=out_specs,
        out_shape=out_shape,
    )(g_ent, g_rec, rels, rel_emb, W1, W2, W3, Wagg, bmat)
    return out.reshape(-1)
